# Initial kernel scaffold; baseline (speedup 1.0000x reference)
import jax
import jax.numpy as jnp
from jax.experimental import pallas as pl


def kernel(scalar_short, scalar_long, vector_short, vector_long, fragment_ids, ln_gamma, ln_beta, W, b):
    raise NotImplementedError("write your pallas kernel here")


# speedup vs baseline: 12.0351x; 12.0351x over previous
import jax
import jax.numpy as jnp
from jax.experimental import pallas as pl
from jax.experimental.pallas import tpu as pltpu

_N = 16384
_H = 512
_NSEG = 256
_MIN_FRAG = 3.0
_CONSISTENCY_FACTOR = 0.03
_INTER_WEIGHT = 0.2
_SCALING = 0.05

_BLK = 2048
_NBLK = _N // _BLK


def _loss_kernel(ids_ref, x_ref, gamma_ref, beta_ref, w_ref, b_ref,
                 out_ref, s1_ref, s2_ref, cnt_ref):
    i = pl.program_id(0)

    @pl.when(i == 0)
    def _init():
        s1_ref[...] = jnp.zeros_like(s1_ref)
        s2_ref[...] = jnp.zeros_like(s2_ref)
        cnt_ref[...] = jnp.zeros_like(cnt_ref)

    x = x_ref[...]
    mu = jnp.mean(x, axis=-1, keepdims=True)
    var = jnp.mean((x - mu) * (x - mu), axis=-1, keepdims=True)
    a = (x - mu) * jax.lax.rsqrt(var + 1e-5)
    a = a * gamma_ref[...] + beta_ref[...]
    y = jax.lax.dot_general(a, w_ref[...], (((1,), (1,)), ((), ())),
                            preferred_element_type=jnp.float32)
    y = y + b_ref[...]
    f = y * jax.nn.sigmoid(y)
    rn = jnp.sqrt(jnp.sum(f * f, axis=1, keepdims=True))
    fn = f / jnp.maximum(rn, 1e-12)

    ids = ids_ref[...].reshape(1, _BLK)
    pt = (jax.lax.broadcasted_iota(jnp.int32, (_NSEG, _BLK), 0) == ids
          ).astype(jnp.float32)
    s1_ref[...] += jax.lax.dot_general(pt, f, (((1,), (0,)), ((), ())),
                                       preferred_element_type=jnp.float32)
    s2_ref[...] += jax.lax.dot_general(pt, fn, (((1,), (0,)), ((), ())),
                                       preferred_element_type=jnp.float32)
    cnt_ref[...] += jnp.sum(pt, axis=1, keepdims=True)

    @pl.when(i == _NBLK - 1)
    def _epilogue():
        s1 = s1_ref[...]
        s2 = s2_ref[...]
        c = cnt_ref[...]
        n1 = jnp.sqrt(jnp.sum(s1 * s1, axis=1, keepdims=True))
        gn = s1 / jnp.maximum(n1, 1e-12)
        ssim = jnp.sum(s2 * gn, axis=1, keepdims=True)
        keep = (c >= _MIN_FRAG).astype(jnp.float32)
        numer = jnp.sum(keep * (c - ssim))
        denom = jnp.sum(keep * c)
        scalar_loss = numer / jnp.maximum(denom, 1e-12)
        t = jnp.sum(gn, axis=0, keepdims=True)
        tr = jnp.sum(gn * gn)
        inter = (jnp.sum(t * t) - tr) / (_NSEG * (_NSEG - 1) + 1e-6)
        total = scalar_loss + _INTER_WEIGHT * inter
        out_ref[...] = (_CONSISTENCY_FACTOR * _SCALING * total).reshape(1, 1)


def kernel(scalar_short, scalar_long, vector_short, vector_long, fragment_ids,
           ln_gamma, ln_beta, W, b):
    ids3 = fragment_ids.reshape(_NBLK, 1, _BLK)
    out = pl.pallas_call(
        _loss_kernel,
        grid=(_NBLK,),
        in_specs=[
            pl.BlockSpec((1, 1, _BLK), lambda i: (i, 0, 0)),
            pl.BlockSpec((_BLK, _H), lambda i: (i, 0)),
            pl.BlockSpec((1, _H), lambda i: (0, 0)),
            pl.BlockSpec((1, _H), lambda i: (0, 0)),
            pl.BlockSpec((_H, _H), lambda i: (0, 0)),
            pl.BlockSpec((1, _H), lambda i: (0, 0)),
        ],
        out_specs=pl.BlockSpec((1, 1), lambda i: (0, 0)),
        out_shape=jax.ShapeDtypeStruct((1, 1), jnp.float32),
        scratch_shapes=[
            pltpu.VMEM((_NSEG, _H), jnp.float32),
            pltpu.VMEM((_NSEG, _H), jnp.float32),
            pltpu.VMEM((_NSEG, 1), jnp.float32),
        ],
        compiler_params=pltpu.CompilerParams(
            dimension_semantics=("arbitrary",),
        ),
    )(ids3, scalar_short, ln_gamma.reshape(1, _H), ln_beta.reshape(1, _H),
      W, b.reshape(1, _H))
    return out.reshape(())
